# baseline (speedup 1.0000x reference)
import jax
import jax.numpy as jnp
from jax import lax
from jax.experimental import pallas as pl
from jax.experimental.pallas import tpu as pltpu

CAP = 16384
NENV = 64
DIM = 32
OBS = 512
HID = 256
RND_OUT = 64
FLAT = NENV * DIM
K = 10
KPAD = 16
EPS = 1e-3
MIN_DIST = 0.008
MAX_SIM = 2.0
C = 1.0
L = 5.0
C_B = 1024
NBLK = CAP // C_B


def _prelude_kernel(obs_ref, w_ide_ref, wp1_ref, wp2_ref, wt1_ref, wt2_ref,
                    emb_ref, mod_ref):
    obs = obs_ref[...]
    emb_ref[...] = jnp.dot(obs, w_ide_ref[...],
                           preferred_element_type=jnp.float32)
    h1 = jnp.maximum(
        jnp.dot(obs, wp1_ref[...], preferred_element_type=jnp.float32), 0.0)
    pred = jnp.dot(h1, wp2_ref[...], preferred_element_type=jnp.float32)
    g1 = jnp.maximum(
        jnp.dot(obs, wt1_ref[...], preferred_element_type=jnp.float32), 0.0)
    tgt = jnp.dot(g1, wt2_ref[...], preferred_element_type=jnp.float32)
    d2 = pred - tgt
    d2 = d2 * d2
    rr = lax.dot_general(jnp.ones((1, RND_OUT), jnp.float32), d2,
                         (((1,), (1,)), ((), ())),
                         preferred_element_type=jnp.float32) / float(RND_OUT)
    mod_ref[...] = jnp.clip(rr + 1.0, 1.0, L)


def _main_kernel(ef_ref, mod_ref, buf_ref, out_ref, s_ref, acc_ref):
    i = pl.program_id(0)

    @pl.when(i == 0)
    def _init():
        rj = lax.broadcasted_iota(jnp.int32, (FLAT, NENV), 0) // DIM
        cn = lax.broadcasted_iota(jnp.int32, (FLAT, NENV), 1)
        s_ref[...] = jnp.where(rj == cn, 1.0, 0.0).astype(jnp.float32)
        acc_ref[...] = jnp.full((KPAD, NENV), jnp.inf, jnp.float32)

    x = buf_ref[...]
    d = x - ef_ref[...]
    sq = d * d
    di = jnp.dot(sq, s_ref[...], preferred_element_type=jnp.float32)

    vals = jnp.concatenate([acc_ref[...], di], axis=0)
    nrow = KPAD + C_B
    rowiota = lax.broadcasted_iota(jnp.int32, (nrow, NENV), 0)
    for kk in range(K):
        m = jnp.min(vals, axis=0, keepdims=True)
        ism = vals == m
        idx = jnp.min(jnp.where(ism, rowiota, nrow), axis=0, keepdims=True)
        vals = jnp.where(rowiota == idx, jnp.inf, vals)
        acc_ref[kk:kk + 1, :] = m

    @pl.when(i == NBLK - 1)
    def _fin():
        accv = acc_ref[...]
        kth = accv[K - 1:K, :]
        avg = jnp.mean(kth)
        scale = jnp.where(avg > 1e-5, 1.0 / avg, 1.0)
        dd = jnp.maximum(accv * scale - MIN_DIST, 0.0)
        kern = EPS / (dd + EPS)
        ksum = jnp.sum(kern, axis=0, keepdims=True)
        s = jnp.sqrt(C + ksum)
        r = jnp.where(s > MAX_SIM, 0.0, 1.0 / s)
        out_ref[...] = r * mod_ref[...] / (1.0 + 1e-5)


def kernel(obs, buffer_data, W_ide, W_pred1, W_pred2, W_tgt1, W_tgt2):
    emb, mod = pl.pallas_call(
        _prelude_kernel,
        in_specs=[
            pl.BlockSpec((NENV, OBS), lambda: (0, 0)),
            pl.BlockSpec((OBS, DIM), lambda: (0, 0)),
            pl.BlockSpec((OBS, HID), lambda: (0, 0)),
            pl.BlockSpec((HID, RND_OUT), lambda: (0, 0)),
            pl.BlockSpec((OBS, HID), lambda: (0, 0)),
            pl.BlockSpec((HID, RND_OUT), lambda: (0, 0)),
        ],
        out_specs=[
            pl.BlockSpec((NENV, DIM), lambda: (0, 0)),
            pl.BlockSpec((1, NENV), lambda: (0, 0)),
        ],
        out_shape=[
            jax.ShapeDtypeStruct((NENV, DIM), jnp.float32),
            jax.ShapeDtypeStruct((1, NENV), jnp.float32),
        ],
    )(obs, W_ide, W_pred1, W_pred2, W_tgt1, W_tgt2)

    ef = emb.reshape(1, FLAT)
    buf2d = buffer_data.reshape(CAP, FLAT)

    out = pl.pallas_call(
        _main_kernel,
        grid=(NBLK,),
        in_specs=[
            pl.BlockSpec((1, FLAT), lambda i: (0, 0)),
            pl.BlockSpec((1, NENV), lambda i: (0, 0)),
            pl.BlockSpec((C_B, FLAT), lambda i: (i, 0)),
        ],
        out_specs=pl.BlockSpec((1, NENV), lambda i: (0, 0)),
        out_shape=jax.ShapeDtypeStruct((1, NENV), jnp.float32),
        scratch_shapes=[
            pltpu.VMEM((FLAT, NENV), jnp.float32),
            pltpu.VMEM((KPAD, NENV), jnp.float32),
        ],
    )(ef, mod, buf2d)
    return out.reshape(NENV)
